# baseline (speedup 1.0000x reference)
import functools

import jax
import jax.numpy as jnp
from jax import lax
from jax.experimental import pallas as pl
from jax.experimental.pallas import tpu as pltpu
from jax.experimental.pallas import tpu_sc as plsc

HIDDEN = 768
MAX_POS = 2048
EPS = 1e-12

NC = 2
NS = 16
NW = NC * NS

CHUNK = 64

TOK_BLK = 2048


def _sc_gather_body(ids_hbm, table_hbm, out_hbm, idx_v, rows_v,
                    sem_g0, sem_g1, sem_s0, sem_s1):
    wid = lax.axis_index("s") * NC + lax.axis_index("c")
    n_chunks = ids_hbm.shape[0] // NW
    base_chunk = wid * n_chunks
    gsems = (sem_g0, sem_g1)
    ssems = (sem_s0, sem_s1)
    pltpu.sync_copy(ids_hbm.at[pl.ds(base_chunk, n_chunks)], idx_v)
    gh = [None, None]
    sh = [None, None]
    gh[0] = pltpu.async_copy(table_hbm.at[idx_v.at[0]], rows_v.at[0], gsems[0])
    for j in range(n_chunks):
        cur = j % 2
        nxt = cur ^ 1
        if j + 1 < n_chunks:
            if sh[nxt] is not None:
                sh[nxt].wait()
            gh[nxt] = pltpu.async_copy(
                table_hbm.at[idx_v.at[j + 1]], rows_v.at[nxt], gsems[nxt])
        gh[cur].wait()
        sh[cur] = pltpu.async_copy(
            rows_v.at[cur],
            out_hbm.at[pl.ds((base_chunk + j) * CHUNK, CHUNK)],
            ssems[cur])
    for h in sh:
        if h is not None:
            h.wait()


def _sc_gather(ids_flat, word_emb):
    tokens = ids_flat.shape[0]
    ids2d = ids_flat.reshape(tokens // CHUNK, CHUNK)
    n_chunks = (tokens // CHUNK) // NW
    mesh = plsc.VectorSubcoreMesh(core_axis_name="c", subcore_axis_name="s")
    return pl.kernel(
        _sc_gather_body,
        out_type=jax.ShapeDtypeStruct((tokens, HIDDEN), jnp.float32),
        mesh=mesh,
        scratch_types=[
            pltpu.VMEM((n_chunks, CHUNK), jnp.int32),
            pltpu.VMEM((2, CHUNK, HIDDEN), jnp.float32),
            pltpu.SemaphoreType.DMA,
            pltpu.SemaphoreType.DMA,
            pltpu.SemaphoreType.DMA,
            pltpu.SemaphoreType.DMA,
        ],
    )(ids2d, word_emb)


def _tc_ln_body(tt_ref, gath_ref, pos_ref, type_ref, w_ref, b_ref, out_ref):
    tt = tt_ref[0][0].reshape(TOK_BLK, 1)
    type0 = type_ref[0:1, :]
    type1 = type_ref[1:2, :]
    e = gath_ref[...] + pos_ref[...] + jnp.where(tt == 0, type0, type1)
    mean = jnp.mean(e, axis=-1, keepdims=True)
    cen = e - mean
    var = jnp.mean(cen * cen, axis=-1, keepdims=True)
    out_ref[...] = w_ref[...] * (cen / jnp.sqrt(var + EPS)) + b_ref[...]


def _tc_ln(tt_flat, gathered, pos_emb, type_emb, ln_weight, ln_bias, batch):
    tokens = gathered.shape[0]
    n_blk = tokens // TOK_BLK
    seq_blocks = n_blk // batch
    tt3d = tt_flat.reshape(n_blk, 1, TOK_BLK)
    return pl.pallas_call(
        _tc_ln_body,
        grid=(seq_blocks, batch),
        in_specs=[
            pl.BlockSpec((1, 1, TOK_BLK), lambda i, j: (j * seq_blocks + i, 0, 0)),
            pl.BlockSpec((TOK_BLK, HIDDEN), lambda i, j: (j * seq_blocks + i, 0)),
            pl.BlockSpec((TOK_BLK, HIDDEN), lambda i, j: (i, 0)),
            pl.BlockSpec((2, HIDDEN), lambda i, j: (0, 0)),
            pl.BlockSpec((1, HIDDEN), lambda i, j: (0, 0)),
            pl.BlockSpec((1, HIDDEN), lambda i, j: (0, 0)),
        ],
        out_specs=pl.BlockSpec((TOK_BLK, HIDDEN), lambda i, j: (j * seq_blocks + i, 0)),
        out_shape=jax.ShapeDtypeStruct((tokens, HIDDEN), jnp.float32),
    )(tt3d, gathered, pos_emb, type_emb,
      ln_weight.reshape(1, HIDDEN), ln_bias.reshape(1, HIDDEN))


def kernel(input_ids, token_type_ids, word_emb, pos_emb, type_emb, ln_weight,
           ln_bias):
    batch, seq = input_ids.shape
    tokens = batch * seq
    ids_flat = input_ids.reshape(tokens).astype(jnp.int32)
    tt_flat = token_type_ids.reshape(tokens).astype(jnp.int32)

    gathered = _sc_gather(ids_flat, word_emb)

    out = _tc_ln(tt_flat, gathered, pos_emb, type_emb, ln_weight, ln_bias,
                 batch)
    return out.reshape(batch, seq, HIDDEN)


# speedup vs baseline: 1.0018x; 1.0018x over previous
import jax
import jax.numpy as jnp
from jax import lax
from jax.experimental import pallas as pl
from jax.experimental.pallas import tpu as pltpu
from jax.experimental.pallas import tpu_sc as plsc

HIDDEN = 768
MAX_POS = 2048
EPS = 1e-12

NC = 2
NS = 16
NW = NC * NS

CHUNK = 64

TOK_BLK = 2048


def _sc_gather_body(ids_hbm, table_hbm, out_hbm, idx_v, rows_v,
                    sem_g0, sem_g1, sem_s0, sem_s1):
    wid = lax.axis_index("s") * NC + lax.axis_index("c")
    n_chunks = ids_hbm.shape[0] // NW
    base_chunk = wid * n_chunks
    gsems = (sem_g0, sem_g1)
    ssems = (sem_s0, sem_s1)
    pltpu.sync_copy(ids_hbm.at[pl.ds(base_chunk, n_chunks)], idx_v)
    gh = [None, None]
    sh = [None, None]
    gh[0] = pltpu.async_copy(table_hbm.at[idx_v.at[0]], rows_v.at[0], gsems[0])
    for j in range(n_chunks):
        cur = j % 2
        nxt = cur ^ 1
        if j + 1 < n_chunks:
            if sh[nxt] is not None:
                sh[nxt].wait()
            gh[nxt] = pltpu.async_copy(
                table_hbm.at[idx_v.at[j + 1]], rows_v.at[nxt], gsems[nxt])
        gh[cur].wait()
        sh[cur] = pltpu.async_copy(
            rows_v.at[cur],
            out_hbm.at[pl.ds((base_chunk + j) * CHUNK, CHUNK)],
            ssems[cur])
    for h in sh:
        if h is not None:
            h.wait()


def _sc_gather(ids_flat, word_emb):
    tokens = ids_flat.shape[0]
    ids2d = ids_flat.reshape(tokens // CHUNK, CHUNK)
    n_chunks = (tokens // CHUNK) // NW
    mesh = plsc.VectorSubcoreMesh(core_axis_name="c", subcore_axis_name="s")
    return pl.kernel(
        _sc_gather_body,
        out_type=jax.ShapeDtypeStruct((tokens, HIDDEN), jnp.float32),
        mesh=mesh,
        scratch_types=[
            pltpu.VMEM((n_chunks, CHUNK), jnp.int32),
            pltpu.VMEM((2, CHUNK, HIDDEN), jnp.float32),
            pltpu.SemaphoreType.DMA,
            pltpu.SemaphoreType.DMA,
            pltpu.SemaphoreType.DMA,
            pltpu.SemaphoreType.DMA,
        ],
    )(ids2d, word_emb)


def _tc_ln_body(tt_ref, gath_ref, pos_ref, type_ref, w_ref, b_ref, out_ref):
    tt = tt_ref[0][0].reshape(TOK_BLK, 1)
    type0 = type_ref[0:1, :]
    type1 = type_ref[1:2, :]
    e = gath_ref[...] + pos_ref[...] + jnp.where(tt == 0, type0, type1)
    mean = jnp.mean(e, axis=-1, keepdims=True)
    cen = e - mean
    var = jnp.mean(cen * cen, axis=-1, keepdims=True)
    out_ref[...] = w_ref[...] * (cen / jnp.sqrt(var + EPS)) + b_ref[...]


def _tc_ln(tt_flat, gathered, pos_emb, type_emb, ln_weight, ln_bias, batch):
    tokens = gathered.shape[0]
    n_blk = tokens // TOK_BLK
    seq_blocks = n_blk // batch
    tt3d = tt_flat.reshape(n_blk, 1, TOK_BLK)
    return pl.pallas_call(
        _tc_ln_body,
        grid=(seq_blocks, batch),
        in_specs=[
            pl.BlockSpec((1, 1, TOK_BLK), lambda i, j: (j * seq_blocks + i, 0, 0)),
            pl.BlockSpec((TOK_BLK, HIDDEN), lambda i, j: (j * seq_blocks + i, 0)),
            pl.BlockSpec((TOK_BLK, HIDDEN), lambda i, j: (i, 0)),
            pl.BlockSpec((2, HIDDEN), lambda i, j: (0, 0)),
            pl.BlockSpec((1, HIDDEN), lambda i, j: (0, 0)),
            pl.BlockSpec((1, HIDDEN), lambda i, j: (0, 0)),
        ],
        out_specs=pl.BlockSpec((TOK_BLK, HIDDEN), lambda i, j: (j * seq_blocks + i, 0)),
        out_shape=jax.ShapeDtypeStruct((tokens, HIDDEN), jnp.float32),
    )(tt3d, gathered, pos_emb, type_emb,
      ln_weight.reshape(1, HIDDEN), ln_bias.reshape(1, HIDDEN))


def kernel(input_ids, token_type_ids, word_emb, pos_emb, type_emb, ln_weight,
           ln_bias):
    batch, seq = input_ids.shape
    tokens = batch * seq
    ids_flat = input_ids.reshape(tokens).astype(jnp.int32)
    tt_flat = token_type_ids.reshape(tokens).astype(jnp.int32)

    gathered = _sc_gather(ids_flat, word_emb)

    out = _tc_ln(tt_flat, gathered, pos_emb, type_emb, ln_weight, ln_bias,
                 batch)
    return out.reshape(batch, seq, HIDDEN)
